# Initial kernel scaffold; baseline (speedup 1.0000x reference)
import jax
import jax.numpy as jnp
from jax.experimental import pallas as pl


def kernel(x, state, edge_index, W_gate, b_gate, W_update, b_update):
    raise NotImplementedError("write your pallas kernel here")


# speedup vs baseline: 34.2056x; 34.2056x over previous
import functools

import jax
import jax.numpy as jnp
from jax import lax
from jax.experimental import pallas as pl
from jax.experimental.pallas import tpu as pltpu
from jax.experimental.pallas import tpu_sc as plsc

B, N, E = 4, 10000, 160000
D_IN, D_H = 128, 64
D_CAT = D_IN + D_H
D_G = 2 * D_H

NC, NS = 2, 16
BPC = B // NC
EPW = E // NS
C = 128
NCH = -(-EPW // C)
EPWP = NCH * C
NP = N + 16
RQ = 624
CQ = RQ // 3
TAIL = N - NS * RQ

_f32 = jnp.float32

_R = 1000
_NB = N // _R
_GRID = (B * N) // _R


def _mm_gate_body(x_ref, s_ref, w_ref, o_ref):
    o_ref[...] = (
        jnp.dot(x_ref[...], w_ref[:D_IN, :], preferred_element_type=_f32)
        + jnp.dot(s_ref[...], w_ref[D_IN:, :], preferred_element_type=_f32)
    )


def _tc_gate_proj(xf, sf, W_gate):
    return pl.pallas_call(
        _mm_gate_body,
        grid=(_GRID,),
        in_specs=[
            pl.BlockSpec((_R, D_IN), lambda i: (i, 0)),
            pl.BlockSpec((_R, D_H), lambda i: (i, 0)),
            pl.BlockSpec((D_CAT, D_G), lambda i: (0, 0)),
        ],
        out_specs=pl.BlockSpec((_R, D_G), lambda i: (i, 0)),
        out_shape=jax.ShapeDtypeStruct((B * N, D_G), _f32),
    )(xf, sf, W_gate)


def _gate_update_body(x_ref, s_ref, a_ref, d_ref, w_ref, bg_ref,
                      t2_ref, r_ref):
    rdeg = 1.0 / jnp.maximum(d_ref[:, 0:1], 1.0)
    halves_t2 = []
    halves_r = []
    for k in range(BPC):
        ur = jax.nn.sigmoid(a_ref[0, k] * rdeg + bg_ref[...])
        u = ur[:, :D_H]
        halves_r.append(ur[:, D_H:])
        halves_t2.append(
            jnp.dot(x_ref[0, k], w_ref[:D_IN, :],
                    preferred_element_type=_f32)
            + jnp.dot(u * s_ref[0, k], w_ref[D_IN:, :],
                      preferred_element_type=_f32)
        )
    t2_ref[0] = jnp.concatenate(halves_t2, axis=-1)
    r_ref[0] = jnp.concatenate(halves_r, axis=-1)


def _tc_gate_update(x4, s4, agg1, deg, W_update, b_gate):
    return pl.pallas_call(
        _gate_update_body,
        grid=(NC * _NB,),
        in_specs=[
            pl.BlockSpec((1, BPC, _R, D_IN),
                         lambda i: (i // _NB, 0, i % _NB, 0)),
            pl.BlockSpec((1, BPC, _R, D_H),
                         lambda i: (i // _NB, 0, i % _NB, 0)),
            pl.BlockSpec((1, BPC, _R, D_G),
                         lambda i: (i // _NB, 0, i % _NB, 0)),
            pl.BlockSpec((_R, 16), lambda i: (i % _NB, 0)),
            pl.BlockSpec((D_CAT, D_H), lambda i: (0, 0)),
            pl.BlockSpec((1, D_G), lambda i: (0, 0)),
        ],
        out_specs=[
            pl.BlockSpec((1, _R, BPC * D_H),
                         lambda i: (i // _NB, i % _NB, 0)),
            pl.BlockSpec((1, _R, BPC * D_H),
                         lambda i: (i // _NB, i % _NB, 0)),
        ],
        out_shape=[
            jax.ShapeDtypeStruct((NC, N, BPC * D_H), _f32),
            jax.ShapeDtypeStruct((NC, N, BPC * D_H), _f32),
        ],
    )(x4, s4, agg1, deg, W_update, b_gate)


def _mix_body(a2_ref, r_ref, s_ref, d_ref, bu_ref, o_ref):
    rdeg = 1.0 / jnp.maximum(d_ref[:, 0:1], 1.0)
    for k in range(BPC):
        hc = jnp.tanh(a2_ref[0][:, k * D_H:(k + 1) * D_H] * rdeg
                      + bu_ref[...])
        r = r_ref[0][:, k * D_H:(k + 1) * D_H]
        o_ref[0, k] = r * s_ref[0, k] + (1.0 - r) * hc


def _tc_mix(agg2, rp, s4, deg, b_update):
    return pl.pallas_call(
        _mix_body,
        grid=(NC * _NB,),
        in_specs=[
            pl.BlockSpec((1, _R, BPC * D_H),
                         lambda i: (i // _NB, i % _NB, 0)),
            pl.BlockSpec((1, _R, BPC * D_H),
                         lambda i: (i // _NB, i % _NB, 0)),
            pl.BlockSpec((1, BPC, _R, D_H),
                         lambda i: (i // _NB, 0, i % _NB, 0)),
            pl.BlockSpec((_R, 16), lambda i: (i % _NB, 0)),
            pl.BlockSpec((1, D_H), lambda i: (0, 0)),
        ],
        out_specs=pl.BlockSpec((1, BPC, _R, D_H),
                               lambda i: (i // _NB, 0, i % _NB, 0)),
        out_shape=jax.ShapeDtypeStruct((NC, BPC, N, D_H), _f32),
    )(agg2, rp, s4, deg, b_update)


_MESH = plsc.VectorSubcoreMesh(core_axis_name="c", subcore_axis_name="s")


def _sc_pass2(t2, src2, dstp, z):
    @functools.partial(
        pl.kernel,
        out_type=jax.ShapeDtypeStruct((NC * N, D_G), _f32),
        mesh=_MESH,
        scratch_types=[
            pltpu.VMEM_SHARED((NP, D_G), _f32),
            pltpu.VMEM((C,), jnp.int32),
            pltpu.VMEM((C,), jnp.int32),
            pltpu.VMEM((C, D_G), _f32),
            pltpu.SemaphoreType.DMA,
        ],
    )
    def k(t2_h, src_h, dst_h, z_h, agg_out, acc, srcl, dstl, buf, sem):
        c = lax.axis_index("c")
        s = lax.axis_index("s")
        last = s == NS - 1
        row0 = s * RQ
        dbase = s * EPWP
        sbase = c * (NS * EPWP) + dbase
        for j in range(3):
            pltpu.sync_copy(z_h.at[pl.ds(j * CQ, CQ)],
                            acc.at[pl.ds(row0 + j * CQ, CQ)])

        @pl.when(last)
        def _():
            pltpu.sync_copy(z_h.at[pl.ds(0, TAIL)],
                            acc.at[pl.ds(N - TAIL, TAIL)])
        plsc.subcore_barrier()

        @pl.loop(0, NCH)
        def _step(g):
            pltpu.sync_copy(src_h.at[pl.ds(sbase + g * C, C)], srcl)
            pltpu.sync_copy(dst_h.at[pl.ds(dbase + g * C, C)], dstl)
            pltpu.async_copy(t2_h.at[srcl], buf, sem).wait()
            pltpu.async_copy(buf, acc.at[dstl], sem, add=True).wait()

        plsc.subcore_barrier()
        for j in range(3):
            pltpu.sync_copy(
                acc.at[pl.ds(row0 + j * CQ, CQ)],
                agg_out.at[pl.ds(c * N + row0 + j * CQ, CQ)])

        @pl.when(last)
        def _():
            pltpu.sync_copy(acc.at[pl.ds(N - TAIL, TAIL)],
                            agg_out.at[pl.ds(c * N + N - TAIL, TAIL)])

    return k(t2, src2, dstp, z)


def kernel(x, state, edge_index, W_gate, b_gate, W_update, b_update):
    xf = x.reshape(B * N, D_IN)
    sf = state.reshape(B * N, D_H)
    x4 = x.reshape(NC, BPC, N, D_IN)
    s4 = state.reshape(NC, BPC, N, D_H)

    pad = EPWP - EPW
    src_t = edge_index[0].reshape(NS, EPW)
    dst_t = edge_index[1].reshape(NS, EPW)
    src_p = jnp.pad(src_t, ((0, 0), (0, pad)))
    dstp = jnp.pad(dst_t, ((0, 0), (0, pad)), constant_values=N).reshape(-1)
    boff = (jnp.arange(B, dtype=jnp.int32) * N)[:, None, None]
    coff = (jnp.arange(NC, dtype=jnp.int32) * N)[:, None, None]
    src2 = (src_p[None] + coff).reshape(-1)
    src2b = (src_p[None] + coff + NC * N).reshape(-1)
    del boff

    z = jnp.zeros((RQ, D_G), _f32)

    t1 = _tc_gate_proj(xf, sf, W_gate)
    aggA = _sc_pass2(t1, src2, dstp, z)
    aggB = _sc_pass2(t1, src2b, dstp, z)
    agg1 = jnp.concatenate([aggA, aggB], axis=0).reshape(NC, BPC, N, D_G)
    deg1 = jnp.bincount(edge_index[1], length=N).astype(_f32)
    deg = jnp.broadcast_to(deg1[:, None], (N, 16))
    t2p, rp = _tc_gate_update(x4, s4, agg1, deg,
                              W_update, b_gate.reshape(1, D_G))
    agg2 = _sc_pass2(t2p.reshape(NC * N, D_G), src2, dstp, z)
    agg2 = agg2.reshape(NC, N, D_G)
    h4 = _tc_mix(agg2, rp, s4, deg, b_update.reshape(1, D_H))
    return h4.reshape(B, N, D_H)
